# baseline (speedup 1.0000x reference)
import functools

import jax
import jax.numpy as jnp
from jax import lax
from jax.experimental import pallas as pl
from jax.experimental.pallas import tpu as pltpu
from jax.experimental.pallas import tpu_sc as plsc

D_MODEL = 128
MAX_REL = 128
SEQ_LEN = 1024
EREV_ROWS = 2 * SEQ_LEN
ROWS_PER_STEP = 16

_SC_INFO = plsc.get_sparse_core_info()
_NC = _SC_INFO.num_cores
_NS = _SC_INFO.num_subcores
_NW = _NC * _NS
_LANES = _SC_INFO.num_lanes
_ROWS_PER_W = EREV_ROWS // _NW


@functools.partial(
    pl.kernel,
    mesh=plsc.VectorSubcoreMesh(core_axis_name="c", subcore_axis_name="s"),
    out_type=jax.ShapeDtypeStruct((EREV_ROWS, D_MODEL), jnp.float32),
    scratch_types=[
        pltpu.VMEM((_ROWS_PER_W,), jnp.int32),
        pltpu.VMEM((_ROWS_PER_W, D_MODEL), jnp.float32),
        pltpu.SemaphoreType.DMA,
    ],
)
def _sc_build_erev(table_hbm, erev_hbm, idx_v, rows_v, sem):
    wid = lax.axis_index("s") * _NC + lax.axis_index("c")
    base = wid * _ROWS_PER_W
    lane = lax.iota(jnp.int32, _LANES)
    for v in range(_ROWS_PER_W // _LANES):
        k = lane + (base + v * _LANES)
        idx = jnp.clip((SEQ_LEN - 1) - k, -MAX_REL, MAX_REL) + MAX_REL
        idx_v[pl.ds(v * _LANES, _LANES)] = idx
    pltpu.async_copy(table_hbm.at[idx_v], rows_v, sem).wait()
    pltpu.sync_copy(rows_v, erev_hbm.at[pl.ds(base, _ROWS_PER_W)])


def _tc_body(erev_ref, out_ref):
    i = pl.program_id(0)
    for r in range(ROWS_PER_STEP):
        row = i * ROWS_PER_STEP + r
        start = (SEQ_LEN - 1) - row
        out_ref[r] = erev_ref[pl.ds(start, SEQ_LEN), :]


def kernel(seq_len, table):
    del seq_len
    erev = _sc_build_erev(table)
    return pl.pallas_call(
        _tc_body,
        grid=(SEQ_LEN // ROWS_PER_STEP,),
        in_specs=[pl.BlockSpec((EREV_ROWS, D_MODEL), lambda i: (0, 0))],
        out_specs=pl.BlockSpec((ROWS_PER_STEP, SEQ_LEN, D_MODEL),
                               lambda i: (i, 0, 0)),
        out_shape=jax.ShapeDtypeStruct((SEQ_LEN, SEQ_LEN, D_MODEL),
                                       jnp.float32),
    )(erev)
